# baseline (speedup 1.0000x reference)
import jax
import jax.numpy as jnp
from jax import lax
from jax.experimental import pallas as pl
from jax.experimental.pallas import tpu as pltpu
from jax.experimental.pallas import tpu_sc as plsc

N = 10000
D = 128
E = 320000
NC = 2
NS = 16
NW = NC * NS
CHUNK = 128
EPW = 10240
EPAD = NW * EPW
ACC_ROWS = 10112
ZROWS = ACC_ROWS // NS
CH_TOT = EPAD // CHUNK
AGG_CORE = 1
CPS = CH_TOT // NS
PH = 80
DEGW = 128
DINVW = 8

import functools


@functools.cache
def _mesh():
    return plsc.VectorSubcoreMesh(core_axis_name="c", subcore_axis_name="s")


NCH = EPW // CHUNK


def _deg_body(dst_hbm, ones_hbm, zeros_hbm, out_hbm, ones_v, didx_all, acc_sh):
    c = lax.axis_index("c")
    s = lax.axis_index("s")
    w = s * NC + c
    pltpu.sync_copy(zeros_hbm, acc_sh.at[pl.ds(s * ZROWS, ZROWS)])
    pltpu.sync_copy(ones_hbm, ones_v)
    pltpu.sync_copy(dst_hbm.at[w], didx_all)
    plsc.subcore_barrier()

    def step(i, carry):
        pltpu.sync_copy(ones_v, acc_sh.at[didx_all.at[i]], add=True)
        return carry

    lax.fori_loop(0, NCH, step, 0)
    plsc.subcore_barrier()
    pltpu.sync_copy(acc_sh.at[pl.ds(s * ZROWS, ZROWS)],
                    out_hbm.at[c, pl.ds(s * ZROWS, ZROWS)])


@functools.cache
def _deg_call():
    return pl.kernel(
        _deg_body,
        out_type=jax.ShapeDtypeStruct((NC, ACC_ROWS, DEGW), jnp.float32),
        mesh=_mesh(),
        scratch_types=[
            pltpu.VMEM((CHUNK, DEGW), jnp.float32),
            pltpu.VMEM((NCH, CHUNK), jnp.int32),
            pltpu.VMEM_SHARED((ACC_ROWS, DEGW), jnp.float32),
        ],
    )


def _agg_body(ua_hbm, ub_hbm, pk_hbm, zeros_hbm, out_hbm,
              pk_all, sidx0, didx0, sidx1, didx1, rows0, rows1,
              acc_sh, sem0, sem1):
    c = lax.axis_index("c")
    s = lax.axis_index("s")
    pltpu.sync_copy(zeros_hbm, acc_sh.at[pl.ds(s * ZROWS, ZROWS)])
    plsc.subcore_barrier()
    m16 = jnp.full((16,), 0xFFFF, jnp.int32)

    def unpack(i, sidx, didx):
        for j in range(CHUNK // 16):
            v = pk_all[i, pl.ds(j * 16, 16)]
            sidx[pl.ds(j * 16, 16)] = v & m16
            didx[pl.ds(j * 16, 16)] = lax.shift_right_logical(v, 16)

    def run(u_hbm):
        base = (c * NS + s) * NCH
        pltpu.sync_copy(pk_hbm.at[pl.ds(base, NCH)], pk_all)

        def prefetch0(i):
            unpack(i, sidx0, didx0)
            pltpu.async_copy(u_hbm.at[sidx0], rows0, sem0)

        def prefetch1(i):
            unpack(i, sidx1, didx1)
            pltpu.async_copy(u_hbm.at[sidx1], rows1, sem1)

        def pair(t, carry):
            i = 2 * t
            prefetch1(i + 1)
            pltpu.make_async_copy(u_hbm.at[sidx0], rows0, sem0).wait()
            pltpu.sync_copy(rows0, acc_sh.at[didx0], add=True)

            @pl.when(t + 1 < NCH // 2)
            def _():
                prefetch0(i + 2)

            pltpu.make_async_copy(u_hbm.at[sidx1], rows1, sem1).wait()
            pltpu.sync_copy(rows1, acc_sh.at[didx1], add=True)
            return carry

        prefetch0(0)
        lax.fori_loop(0, NCH // 2, pair, 0)

    @pl.when(c == 0)
    def _():
        run(ua_hbm)

    @pl.when(c == 1)
    def _():
        run(ub_hbm)

    plsc.subcore_barrier()
    pltpu.sync_copy(acc_sh.at[pl.ds(s * ZROWS, ZROWS)],
                    out_hbm.at[c, pl.ds(s * ZROWS, ZROWS)])


@functools.cache
def _agg_call():
    return pl.kernel(
        _agg_body,
        out_type=jax.ShapeDtypeStruct((NC, ACC_ROWS, D), jnp.float32),
        mesh=_mesh(),
        scratch_types=[
            pltpu.VMEM((NCH, CHUNK), jnp.int32),
            pltpu.VMEM((CHUNK,), jnp.int32),
            pltpu.VMEM((CHUNK,), jnp.int32),
            pltpu.VMEM((CHUNK,), jnp.int32),
            pltpu.VMEM((CHUNK,), jnp.int32),
            pltpu.VMEM((CHUNK, D), jnp.float32),
            pltpu.VMEM((CHUNK, D), jnp.float32),
            pltpu.VMEM_SHARED((ACC_ROWS, D), jnp.float32),
            pltpu.SemaphoreType.DMA,
            pltpu.SemaphoreType.DMA,
        ],
    )


_R = 2000
_G = N // _R


def _tc1_body(deg_ref, x_ref, w_ref, u_ref, ub_ref, dinv_ref):
    deg = deg_ref[0] + deg_ref[1] + 1.0
    dinv = lax.rsqrt(deg)
    dinv_ref[...] = dinv[:, :DINVW]
    h = jnp.dot(x_ref[...], w_ref[...], preferred_element_type=jnp.float32)
    u = h * dinv[:, 0:1]
    u_ref[...] = u
    ub_ref[...] = u


_tc1_call = pl.pallas_call(
    _tc1_body,
    grid=(_G,),
    in_specs=[
        pl.BlockSpec((NC, _R, DEGW), lambda i: (0, i, 0)),
        pl.BlockSpec((_R, D), lambda i: (i, 0)),
        pl.BlockSpec((D, D), lambda i: (0, 0)),
    ],
    out_specs=[
        pl.BlockSpec((_R, D), lambda i: (i, 0)),
        pl.BlockSpec((_R, D), lambda i: (i, 0)),
        pl.BlockSpec((_R, DINVW), lambda i: (i, 0)),
    ],
    out_shape=[
        jax.ShapeDtypeStruct((N, D), jnp.float32),
        jax.ShapeDtypeStruct((N, D), jnp.float32),
        jax.ShapeDtypeStruct((N, DINVW), jnp.float32),
    ],
)


def _tc2_body(p_ref, u_ref, dinv_ref, b_ref, w_ref, o_ref, ob_ref):
    dinv = dinv_ref[:, 0:1]
    agg = (p_ref[0] + p_ref[1] + u_ref[...]) * dinv
    x2 = jnp.maximum(agg + b_ref[...], 0.0)
    o = jnp.dot(x2, w_ref[...], preferred_element_type=jnp.float32) * dinv
    o_ref[...] = o
    ob_ref[...] = o


_tc2_call = pl.pallas_call(
    _tc2_body,
    grid=(_G,),
    in_specs=[
        pl.BlockSpec((NC, _R, D), lambda i: (0, i, 0)),
        pl.BlockSpec((_R, D), lambda i: (i, 0)),
        pl.BlockSpec((_R, DINVW), lambda i: (i, 0)),
        pl.BlockSpec((1, D), lambda i: (0, 0)),
        pl.BlockSpec((D, D), lambda i: (0, 0)),
    ],
    out_specs=[
        pl.BlockSpec((_R, D), lambda i: (i, 0)),
        pl.BlockSpec((_R, D), lambda i: (i, 0)),
    ],
    out_shape=[
        jax.ShapeDtypeStruct((N, D), jnp.float32),
        jax.ShapeDtypeStruct((N, D), jnp.float32),
    ],
)


def _tc3_body(q_ref, u_ref, dinv_ref, b2_ref, w3_ref, b3_ref, o_ref):
    dinv = dinv_ref[:, 0:1]
    agg = (q_ref[0] + q_ref[1] + u_ref[...]) * dinv
    x3 = jnp.maximum(agg + b2_ref[...], 0.0)
    o_ref[...] = jnp.dot(x3, w3_ref[...],
                         preferred_element_type=jnp.float32) + b3_ref[...]


_tc3_call = pl.pallas_call(
    _tc3_body,
    grid=(_G,),
    in_specs=[
        pl.BlockSpec((NC, _R, D), lambda i: (0, i, 0)),
        pl.BlockSpec((_R, D), lambda i: (i, 0)),
        pl.BlockSpec((_R, DINVW), lambda i: (i, 0)),
        pl.BlockSpec((1, D), lambda i: (0, 0)),
        pl.BlockSpec((D, D // 2), lambda i: (0, 0)),
        pl.BlockSpec((1, D // 2), lambda i: (0, 0)),
    ],
    out_specs=pl.BlockSpec((_R, D // 2), lambda i: (i, 0)),
    out_shape=jax.ShapeDtypeStruct((N, D // 2), jnp.float32),
)


def kernel(x, edge_index, W1, b1, W2, b2, W3, b3):
    src = edge_index[0].astype(jnp.int32)
    dst = edge_index[1].astype(jnp.int32)
    pad = EPAD - E
    src_p = jnp.concatenate([src, jnp.zeros((pad,), jnp.int32)])
    dst_p = jnp.concatenate([dst, jnp.full((pad,), N, jnp.int32)])
    pk = (src_p | (dst_p << 16)).reshape(CH_TOT, CHUNK)
    dst_p = dst_p.reshape(NW, NCH, CHUNK)
    ones8 = jnp.ones((CHUNK, DEGW), jnp.float32)
    zeros8 = jnp.zeros((ZROWS, DEGW), jnp.float32)
    zerosD = jnp.zeros((ZROWS, D), jnp.float32)

    deg8 = _deg_call()(dst_p, ones8, zeros8)
    u1, u1b, dinv8 = _tc1_call(deg8, x, W1)
    s1 = _agg_call()(u1, u1b, pk, zerosD)
    u2, u2b = _tc2_call(s1, u1, dinv8, b1.reshape(1, D), W2)
    s2 = _agg_call()(u2, u2b, pk, zerosD)
    return _tc3_call(s2, u2, dinv8, b2.reshape(1, D), W3.reshape(D, D // 2),
                     b3.reshape(1, D // 2))


# speedup vs baseline: 1.0794x; 1.0794x over previous
import jax
import jax.numpy as jnp
from jax import lax
from jax.experimental import pallas as pl
from jax.experimental.pallas import tpu as pltpu
from jax.experimental.pallas import tpu_sc as plsc

N = 10000
D = 128
E = 320000
NC = 2
NS = 16
NW = NC * NS
CHUNK = 128
EPW = 10240
EPAD = NW * EPW
ACC_ROWS = 10112
ZROWS = ACC_ROWS // NS
CH_TOT = EPAD // CHUNK
AGG_CORE = 1
CPS = CH_TOT // NS
PH = 80
DEGW = 128
DINVW = 8

import functools


@functools.cache
def _mesh():
    return plsc.VectorSubcoreMesh(core_axis_name="c", subcore_axis_name="s")


NCH = EPW // CHUNK


def _deg_body(dst_hbm, ones_hbm, zeros_hbm, out_hbm, ones_v, didx_all, acc_sh):
    c = lax.axis_index("c")
    s = lax.axis_index("s")
    w = s * NC + c
    pltpu.sync_copy(zeros_hbm, acc_sh.at[pl.ds(s * ZROWS, ZROWS)])
    pltpu.sync_copy(ones_hbm, ones_v)
    pltpu.sync_copy(dst_hbm.at[w], didx_all)
    plsc.subcore_barrier()

    def step(i, carry):
        pltpu.sync_copy(ones_v, acc_sh.at[didx_all.at[i]], add=True)
        return carry

    lax.fori_loop(0, NCH, step, 0)
    plsc.subcore_barrier()
    pltpu.sync_copy(acc_sh.at[pl.ds(s * ZROWS, ZROWS)],
                    out_hbm.at[c, pl.ds(s * ZROWS, ZROWS)])


@functools.cache
def _deg_call():
    return pl.kernel(
        _deg_body,
        out_type=jax.ShapeDtypeStruct((NC, ACC_ROWS, DEGW), jnp.float32),
        mesh=_mesh(),
        scratch_types=[
            pltpu.VMEM((CHUNK, DEGW), jnp.float32),
            pltpu.VMEM((NCH, CHUNK), jnp.int32),
            pltpu.VMEM_SHARED((ACC_ROWS, DEGW), jnp.float32),
        ],
    )


def _agg_body(u_hbm, pk_hbm, zeros_hbm, out_hbm,
              pk_all, sidx0, didx0, sidx1, didx1, rows0, rows1,
              acc_sh, sem0, sem1):
    c = lax.axis_index("c")
    s = lax.axis_index("s")
    w = s * NC + c
    pltpu.sync_copy(zeros_hbm, acc_sh.at[pl.ds(s * ZROWS, ZROWS)])
    pltpu.sync_copy(pk_hbm.at[pl.ds(w * NCH, NCH)], pk_all)
    plsc.subcore_barrier()
    m16 = jnp.full((16,), 0xFFFF, jnp.int32)

    def unpack(i, sidx, didx):
        for j in range(CHUNK // 16):
            v = pk_all[i, pl.ds(j * 16, 16)]
            sidx[pl.ds(j * 16, 16)] = v & m16
            didx[pl.ds(j * 16, 16)] = lax.shift_right_logical(v, 16)

    def prefetch0(i):
        unpack(i, sidx0, didx0)
        pltpu.async_copy(u_hbm.at[sidx0], rows0, sem0)

    def prefetch1(i):
        unpack(i, sidx1, didx1)
        pltpu.async_copy(u_hbm.at[sidx1], rows1, sem1)

    def pair(t, carry):
        i = 2 * t
        prefetch1(i + 1)
        pltpu.make_async_copy(u_hbm.at[sidx0], rows0, sem0).wait()
        pltpu.sync_copy(rows0, acc_sh.at[didx0], add=True)

        @pl.when(t + 1 < NCH // 2)
        def _():
            prefetch0(i + 2)

        pltpu.make_async_copy(u_hbm.at[sidx1], rows1, sem1).wait()
        pltpu.sync_copy(rows1, acc_sh.at[didx1], add=True)
        return carry

    prefetch0(0)
    lax.fori_loop(0, NCH // 2, pair, 0)
    plsc.subcore_barrier()
    pltpu.sync_copy(acc_sh.at[pl.ds(s * ZROWS, ZROWS)],
                    out_hbm.at[c, pl.ds(s * ZROWS, ZROWS)])


@functools.cache
def _agg_call():
    return pl.kernel(
        _agg_body,
        out_type=jax.ShapeDtypeStruct((NC, ACC_ROWS, D), jnp.float32),
        mesh=_mesh(),
        scratch_types=[
            pltpu.VMEM((NCH, CHUNK), jnp.int32),
            pltpu.VMEM((CHUNK,), jnp.int32),
            pltpu.VMEM((CHUNK,), jnp.int32),
            pltpu.VMEM((CHUNK,), jnp.int32),
            pltpu.VMEM((CHUNK,), jnp.int32),
            pltpu.VMEM((CHUNK, D), jnp.float32),
            pltpu.VMEM((CHUNK, D), jnp.float32),
            pltpu.VMEM_SHARED((ACC_ROWS, D), jnp.float32),
            pltpu.SemaphoreType.DMA,
            pltpu.SemaphoreType.DMA,
        ],
    )


_R = 2000
_G = N // _R


def _tc1_body(deg_ref, x_ref, w_ref, u_ref, dinv_ref):
    deg = deg_ref[0] + deg_ref[1] + 1.0
    dinv = lax.rsqrt(deg)
    dinv_ref[...] = dinv[:, :DINVW]
    h = jnp.dot(x_ref[...], w_ref[...], preferred_element_type=jnp.float32)
    u_ref[...] = h * dinv[:, 0:1]


_tc1_call = pl.pallas_call(
    _tc1_body,
    grid=(_G,),
    in_specs=[
        pl.BlockSpec((NC, _R, DEGW), lambda i: (0, i, 0)),
        pl.BlockSpec((_R, D), lambda i: (i, 0)),
        pl.BlockSpec((D, D), lambda i: (0, 0)),
    ],
    out_specs=[
        pl.BlockSpec((_R, D), lambda i: (i, 0)),
        pl.BlockSpec((_R, DINVW), lambda i: (i, 0)),
    ],
    out_shape=[
        jax.ShapeDtypeStruct((N, D), jnp.float32),
        jax.ShapeDtypeStruct((N, DINVW), jnp.float32),
    ],
)


def _tc2_body(p_ref, u_ref, dinv_ref, b_ref, w_ref, o_ref):
    dinv = dinv_ref[:, 0:1]
    agg = (p_ref[0] + p_ref[1] + u_ref[...]) * dinv
    x2 = jnp.maximum(agg + b_ref[...], 0.0)
    o_ref[...] = jnp.dot(x2, w_ref[...],
                         preferred_element_type=jnp.float32) * dinv


_tc2_call = pl.pallas_call(
    _tc2_body,
    grid=(_G,),
    in_specs=[
        pl.BlockSpec((NC, _R, D), lambda i: (0, i, 0)),
        pl.BlockSpec((_R, D), lambda i: (i, 0)),
        pl.BlockSpec((_R, DINVW), lambda i: (i, 0)),
        pl.BlockSpec((1, D), lambda i: (0, 0)),
        pl.BlockSpec((D, D), lambda i: (0, 0)),
    ],
    out_specs=pl.BlockSpec((_R, D), lambda i: (i, 0)),
    out_shape=jax.ShapeDtypeStruct((N, D), jnp.float32),
)


def _tc3_body(q_ref, u_ref, dinv_ref, b2_ref, w3_ref, b3_ref, o_ref):
    dinv = dinv_ref[:, 0:1]
    agg = (q_ref[0] + q_ref[1] + u_ref[...]) * dinv
    x3 = jnp.maximum(agg + b2_ref[...], 0.0)
    o_ref[...] = jnp.dot(x3, w3_ref[...],
                         preferred_element_type=jnp.float32) + b3_ref[...]


_tc3_call = pl.pallas_call(
    _tc3_body,
    grid=(_G,),
    in_specs=[
        pl.BlockSpec((NC, _R, D), lambda i: (0, i, 0)),
        pl.BlockSpec((_R, D), lambda i: (i, 0)),
        pl.BlockSpec((_R, DINVW), lambda i: (i, 0)),
        pl.BlockSpec((1, D), lambda i: (0, 0)),
        pl.BlockSpec((D, D // 2), lambda i: (0, 0)),
        pl.BlockSpec((1, D // 2), lambda i: (0, 0)),
    ],
    out_specs=pl.BlockSpec((_R, D // 2), lambda i: (i, 0)),
    out_shape=jax.ShapeDtypeStruct((N, D // 2), jnp.float32),
)


def kernel(x, edge_index, W1, b1, W2, b2, W3, b3):
    src = edge_index[0].astype(jnp.int32)
    dst = edge_index[1].astype(jnp.int32)
    pad = EPAD - E
    src_p = jnp.concatenate([src, jnp.zeros((pad,), jnp.int32)])
    dst_p = jnp.concatenate([dst, jnp.full((pad,), N, jnp.int32)])
    pk = (src_p | (dst_p << 16)).reshape(CH_TOT, CHUNK)
    dst_p = dst_p.reshape(NW, NCH, CHUNK)
    ones8 = jnp.ones((CHUNK, DEGW), jnp.float32)
    zeros8 = jnp.zeros((ZROWS, DEGW), jnp.float32)
    zerosD = jnp.zeros((ZROWS, D), jnp.float32)

    deg8 = _deg_call()(dst_p, ones8, zeros8)
    u1, dinv8 = _tc1_call(deg8, x, W1)
    s1 = _agg_call()(u1, pk, zerosD)
    u2 = _tc2_call(s1, u1, dinv8, b1.reshape(1, D), W2)
    s2 = _agg_call()(u2, pk, zerosD)
    return _tc3_call(s2, u2, dinv8, b2.reshape(1, D), W3.reshape(D, D // 2),
                     b3.reshape(1, D // 2))


# speedup vs baseline: 1.2032x; 1.1147x over previous
import jax
import jax.numpy as jnp
from jax import lax
from jax.experimental import pallas as pl
from jax.experimental.pallas import tpu as pltpu
from jax.experimental.pallas import tpu_sc as plsc

N = 10000
D = 128
E = 320000
NC = 2
NS = 16
NW = NC * NS
CHUNK = 128
EPW = 10240
EPAD = NW * EPW
ACC_ROWS = 10240
ZROWS = ACC_ROWS // NS
CH_TOT = EPAD // CHUNK
AGG_CORE = 1
CPS = CH_TOT // NS
PH = 80
DEGW = 128
DINVW = 8

import functools


@functools.cache
def _mesh():
    return plsc.VectorSubcoreMesh(core_axis_name="c", subcore_axis_name="s")


NCH = EPW // CHUNK


def _deg_body(dst_hbm, ones_hbm, zeros_hbm, out_hbm, ones_v, didx_all, acc_sh):
    c = lax.axis_index("c")
    s = lax.axis_index("s")
    w = s * NC + c
    pltpu.sync_copy(zeros_hbm, acc_sh.at[pl.ds(s * ZROWS, ZROWS)])
    pltpu.sync_copy(ones_hbm, ones_v)
    pltpu.sync_copy(dst_hbm.at[w], didx_all)
    plsc.subcore_barrier()

    def step(i, carry):
        pltpu.sync_copy(ones_v, acc_sh.at[didx_all.at[i]], add=True)
        return carry

    lax.fori_loop(0, NCH, step, 0)
    plsc.subcore_barrier()
    pltpu.sync_copy(acc_sh.at[pl.ds(s * ZROWS, ZROWS)],
                    out_hbm.at[c, pl.ds(s * ZROWS, ZROWS)])


@functools.cache
def _deg_call():
    return pl.kernel(
        _deg_body,
        out_type=jax.ShapeDtypeStruct((NC, ACC_ROWS, DEGW), jnp.float32),
        mesh=_mesh(),
        scratch_types=[
            pltpu.VMEM((CHUNK, DEGW), jnp.float32),
            pltpu.VMEM((NCH, CHUNK), jnp.int32),
            pltpu.VMEM_SHARED((ACC_ROWS, DEGW), jnp.float32),
        ],
    )


def _agg_body(u_hbm, pk_hbm, zeros_hbm, out_hbm,
              pk_all, sidx0, didx0, sidx1, didx1, rows0, rows1,
              acc_sh, sem0, sem1):
    c = lax.axis_index("c")
    s = lax.axis_index("s")
    w = s * NC + c
    pltpu.sync_copy(zeros_hbm, acc_sh.at[pl.ds(s * ZROWS, ZROWS)])
    pltpu.sync_copy(pk_hbm.at[w], pk_all)
    plsc.subcore_barrier()
    m16 = jnp.full((16,), 0xFFFF, jnp.int32)

    def unpack(i, sidx, didx):
        for j in range(CHUNK // 16):
            v = pk_all[i, pl.ds(j * 16, 16)]
            sidx[pl.ds(j * 16, 16)] = v & m16
            didx[pl.ds(j * 16, 16)] = lax.shift_right_logical(v, 16)

    def prefetch0(i):
        unpack(i, sidx0, didx0)
        pltpu.async_copy(u_hbm.at[sidx0], rows0, sem0)

    def prefetch1(i):
        unpack(i, sidx1, didx1)
        pltpu.async_copy(u_hbm.at[sidx1], rows1, sem1)

    def pair(t, carry):
        i = 2 * t
        prefetch1(i + 1)
        pltpu.make_async_copy(u_hbm.at[sidx0], rows0, sem0).wait()
        pltpu.sync_copy(rows0, acc_sh.at[didx0], add=True)

        @pl.when(t + 1 < NCH // 2)
        def _():
            prefetch0(i + 2)

        pltpu.make_async_copy(u_hbm.at[sidx1], rows1, sem1).wait()
        pltpu.sync_copy(rows1, acc_sh.at[didx1], add=True)
        return carry

    prefetch0(0)
    lax.fori_loop(0, NCH // 2, pair, 0)
    plsc.subcore_barrier()
    pltpu.sync_copy(acc_sh.at[pl.ds(s * ZROWS, ZROWS)],
                    out_hbm.at[c, pl.ds(s * ZROWS, ZROWS)])


@functools.cache
def _agg_call():
    return pl.kernel(
        _agg_body,
        out_type=jax.ShapeDtypeStruct((NC, ACC_ROWS, D), jnp.float32),
        mesh=_mesh(),
        scratch_types=[
            pltpu.VMEM((NCH, CHUNK), jnp.int32),
            pltpu.VMEM((CHUNK,), jnp.int32),
            pltpu.VMEM((CHUNK,), jnp.int32),
            pltpu.VMEM((CHUNK,), jnp.int32),
            pltpu.VMEM((CHUNK,), jnp.int32),
            pltpu.VMEM((CHUNK, D), jnp.float32),
            pltpu.VMEM((CHUNK, D), jnp.float32),
            pltpu.VMEM_SHARED((ACC_ROWS, D), jnp.float32),
            pltpu.SemaphoreType.DMA,
            pltpu.SemaphoreType.DMA,
        ],
    )


_R = 2000
_G = N // _R


def _tc1_body(deg_ref, x_ref, w_ref, u_ref, dinv_ref):
    deg = deg_ref[0] + deg_ref[1] + 1.0
    dinv = lax.rsqrt(deg)
    dinv_ref[...] = dinv[:, :DINVW]
    h = jnp.dot(x_ref[...], w_ref[...], preferred_element_type=jnp.float32)
    u_ref[...] = h * dinv[:, 0:1]


_tc1_call = pl.pallas_call(
    _tc1_body,
    grid=(_G,),
    in_specs=[
        pl.BlockSpec((NC, _R, DEGW), lambda i: (0, i, 0)),
        pl.BlockSpec((_R, D), lambda i: (i, 0)),
        pl.BlockSpec((D, D), lambda i: (0, 0)),
    ],
    out_specs=[
        pl.BlockSpec((_R, D), lambda i: (i, 0)),
        pl.BlockSpec((_R, DINVW), lambda i: (i, 0)),
    ],
    out_shape=[
        jax.ShapeDtypeStruct((N, D), jnp.float32),
        jax.ShapeDtypeStruct((N, DINVW), jnp.float32),
    ],
)


def _tc2_body(p_ref, u_ref, dinv_ref, b_ref, w_ref, o_ref):
    dinv = dinv_ref[:, 0:1]
    agg = (p_ref[0] + p_ref[1] + u_ref[...]) * dinv
    x2 = jnp.maximum(agg + b_ref[...], 0.0)
    o_ref[...] = jnp.dot(x2, w_ref[...],
                         preferred_element_type=jnp.float32) * dinv


_tc2_call = pl.pallas_call(
    _tc2_body,
    grid=(_G,),
    in_specs=[
        pl.BlockSpec((NC, _R, D), lambda i: (0, i, 0)),
        pl.BlockSpec((_R, D), lambda i: (i, 0)),
        pl.BlockSpec((_R, DINVW), lambda i: (i, 0)),
        pl.BlockSpec((1, D), lambda i: (0, 0)),
        pl.BlockSpec((D, D), lambda i: (0, 0)),
    ],
    out_specs=pl.BlockSpec((_R, D), lambda i: (i, 0)),
    out_shape=jax.ShapeDtypeStruct((N, D), jnp.float32),
)


def _tc3_body(q_ref, u_ref, dinv_ref, b2_ref, w3_ref, b3_ref, o_ref):
    dinv = dinv_ref[:, 0:1]
    agg = (q_ref[0] + q_ref[1] + u_ref[...]) * dinv
    x3 = jnp.maximum(agg + b2_ref[...], 0.0)
    o_ref[...] = jnp.dot(x3, w3_ref[...],
                         preferred_element_type=jnp.float32) + b3_ref[...]


_tc3_call = pl.pallas_call(
    _tc3_body,
    grid=(_G,),
    in_specs=[
        pl.BlockSpec((NC, _R, D), lambda i: (0, i, 0)),
        pl.BlockSpec((_R, D), lambda i: (i, 0)),
        pl.BlockSpec((_R, DINVW), lambda i: (i, 0)),
        pl.BlockSpec((1, D), lambda i: (0, 0)),
        pl.BlockSpec((D, D // 2), lambda i: (0, 0)),
        pl.BlockSpec((1, D // 2), lambda i: (0, 0)),
    ],
    out_specs=pl.BlockSpec((_R, D // 2), lambda i: (i, 0)),
    out_shape=jax.ShapeDtypeStruct((N, D // 2), jnp.float32),
)


def kernel(x, edge_index, W1, b1, W2, b2, W3, b3):
    src = edge_index[0].astype(jnp.int32)
    dst = edge_index[1].astype(jnp.int32)
    pad = EPAD - E
    src_p = jnp.concatenate([src, jnp.zeros((pad,), jnp.int32)])
    dst_p = jnp.concatenate([dst, jnp.full((pad,), N, jnp.int32)])
    pk = (src_p | (dst_p << 16)).reshape(NW, NCH, CHUNK)
    dst_p = dst_p.reshape(NW, NCH, CHUNK)
    ones8 = jnp.ones((CHUNK, DEGW), jnp.float32)
    zeros8 = jnp.zeros((ZROWS, DEGW), jnp.float32)
    zerosD = jnp.zeros((ZROWS, D), jnp.float32)

    deg8 = _deg_call()(dst_p, ones8, zeros8)
    u1, dinv8 = _tc1_call(deg8, x, W1)
    s1 = _agg_call()(u1, pk, zerosD)
    u2 = _tc2_call(s1, u1, dinv8, b1.reshape(1, D), W2)
    s2 = _agg_call()(u2, pk, zerosD)
    return _tc3_call(s2, u2, dinv8, b2.reshape(1, D), W3.reshape(D, D // 2),
                     b3.reshape(1, D // 2))
